# baseline (speedup 1.0000x reference)
import jax
import jax.numpy as jnp
from jax.experimental import pallas as pl

_ROWS = 8


def _hardmax_block(x_ref, o_ref):
    b = x_ref[...]
    n = b.shape[1]
    m = jnp.max(b, axis=1, keepdims=True)
    iota = jax.lax.broadcasted_iota(jnp.int32, b.shape, 1)
    idx = jnp.min(jnp.where(b == m, iota, jnp.int32(n)), axis=1, keepdims=True)
    o_ref[...] = (iota == idx).astype(jnp.int32)


def kernel(x):
    B, R, N = x.shape
    rows = B * R
    xf = x.reshape(rows, N)
    out = pl.pallas_call(
        _hardmax_block,
        grid=(rows // _ROWS,),
        in_specs=[pl.BlockSpec((_ROWS, N), lambda i: (i, 0))],
        out_specs=pl.BlockSpec((_ROWS, N), lambda i: (i, 0)),
        out_shape=jax.ShapeDtypeStruct((rows, N), jnp.int32),
    )(xf)
    return out.reshape(B, R, N)
